# baseline (speedup 1.0000x reference)
import functools

import jax
import jax.numpy as jnp
from jax import lax
from jax.experimental import pallas as pl
from jax.experimental.pallas import tpu as pltpu
from jax.experimental.pallas import tpu_sc as plsc

D_MODEL = 768
MAX_SEQ = 1024
NUM_CORES = 2
NUM_SUBCORES = 16
NW = NUM_CORES * NUM_SUBCORES
LANES = 16
GROUPS = D_MODEL // LANES

_mesh = plsc.VectorSubcoreMesh(core_axis_name="c", subcore_axis_name="s")


@functools.cache
def _build(total_tokens: int, chunk: int):
    tok_per_w = total_tokens // NW
    n_chunks = tok_per_w // chunk

    @functools.partial(
        pl.kernel,
        mesh=_mesh,
        out_type=jax.ShapeDtypeStruct((total_tokens, D_MODEL), jnp.float32),
        scratch_types=[
            pltpu.VMEM((chunk,), jnp.int32),
            pltpu.VMEM((chunk, D_MODEL), jnp.float32),
            pltpu.VMEM((chunk, D_MODEL), jnp.float32),
            pltpu.VMEM_SHARED((MAX_SEQ, D_MODEL), jnp.float32),
            pltpu.SemaphoreType.DMA,
        ],
    )
    def emb(x_hbm, tok_hbm, pos_hbm, out_hbm, idx_v, rows_v, pos_v, pos_sh, sem):
        sid = lax.axis_index("s")
        wid = sid * NUM_CORES + lax.axis_index("c")
        base0 = pl.multiple_of(wid * tok_per_w, tok_per_w)

        @pl.when(sid == 0)
        def _fill():
            pltpu.sync_copy(pos_hbm, pos_sh)

        plsc.subcore_barrier()

        def chunk_body(k, _):
            base = base0 + k * chunk
            p0 = lax.rem(k * chunk, MAX_SEQ)
            pltpu.sync_copy(x_hbm.at[pl.ds(base, chunk)], idx_v)
            gather = pltpu.async_copy(tok_hbm.at[idx_v], rows_v, sem)
            pltpu.sync_copy(pos_sh.at[pl.ds(p0, chunk)], pos_v)
            gather.wait()

            def row_body(r, _):
                for g in range(GROUPS):
                    sl = pl.ds(g * LANES, LANES)
                    rows_v[r, sl] = rows_v[r, sl] + pos_v[r, sl]
                return 0

            lax.fori_loop(0, chunk, row_body, 0)
            pltpu.sync_copy(rows_v, out_hbm.at[pl.ds(base, chunk)])
            return 0

        lax.fori_loop(0, n_chunks, chunk_body, 0)

    return emb


def kernel(x, tok_table, pos_table):
    B, T = x.shape
    total = B * T
    emb = _build(total, 32)
    out = emb(x.reshape(total).astype(jnp.int32), tok_table, pos_table)
    return out.reshape(B, T, D_MODEL)


# speedup vs baseline: 2.0429x; 2.0429x over previous
import functools

import jax
import jax.numpy as jnp
from jax import lax
from jax.experimental import pallas as pl
from jax.experimental.pallas import tpu as pltpu
from jax.experimental.pallas import tpu_sc as plsc

D_MODEL = 768
NUM_CORES = 2
NUM_SUBCORES = 16
NW = NUM_CORES * NUM_SUBCORES
LANES = 16
GROUPS = D_MODEL // LANES
NBUF = 4
PREFETCH = 2

_mesh = plsc.VectorSubcoreMesh(core_axis_name="c", subcore_axis_name="s")


@functools.cache
def _build(n_seq: int, seq_len: int):
    ppw = seq_len // NW
    total = n_seq * seq_len

    @functools.partial(
        pl.kernel,
        mesh=_mesh,
        out_type=jax.ShapeDtypeStruct((total, D_MODEL), jnp.float32),
        scratch_types=[
            pltpu.VMEM((n_seq * ppw,), jnp.int32),
            pltpu.VMEM((ppw, D_MODEL), jnp.float32),
        ]
        + [pltpu.VMEM((ppw, D_MODEL), jnp.float32) for _ in range(NBUF)]
        + [pltpu.SemaphoreType.DMA for _ in range(2 * NBUF)],
    )
    def emb(x_hbm, tok_hbm, pos_hbm, out_hbm, idx_all, pos_v, *bufs_sems):
        rows = bufs_sems[:NBUF]
        gsem = bufs_sems[NBUF:2 * NBUF]
        osem = bufs_sems[2 * NBUF:]
        wid = lax.axis_index("s") * NUM_CORES + lax.axis_index("c")
        p_lo = pl.multiple_of(wid * ppw, ppw)

        def idx_body(s, _):
            pltpu.async_copy(
                x_hbm.at[pl.ds(s * seq_len + p_lo, ppw)],
                idx_all.at[pl.ds(s * ppw, ppw)],
                gsem[0],
            )
            return 0

        lax.fori_loop(0, n_seq, idx_body, 0)
        pltpu.sync_copy(pos_hbm.at[pl.ds(p_lo, ppw)], pos_v)
        pltpu.make_async_copy(x_hbm.at[pl.ds(0, n_seq * ppw)],
                              idx_all, gsem[0]).wait()

        def start_gather(b, k):
            pltpu.async_copy(
                tok_hbm.at[idx_all.at[pl.ds(k * ppw, ppw)]], rows[b], gsem[b]
            )

        def wait_gather(b):
            pltpu.make_async_copy(
                tok_hbm.at[idx_all.at[pl.ds(0, ppw)]], rows[b], gsem[b]
            ).wait()

        def start_out(b, k):
            pltpu.async_copy(rows[b], out_hbm.at[pl.ds(k * seq_len + p_lo, ppw)], osem[b])

        def wait_out(b):
            pltpu.make_async_copy(rows[b], out_hbm.at[pl.ds(0, ppw)], osem[b]).wait()

        for k0 in range(PREFETCH):
            start_gather(k0, k0)

        def body(j, _):
            for b in range(NBUF):
                k = NBUF * j + b
                wait_gather(b)

                def row_body(r, _):
                    for g in range(GROUPS):
                        sl = pl.ds(g * LANES, LANES)
                        rows[b][r, sl] = rows[b][r, sl] + pos_v[r, sl]
                    return 0

                lax.fori_loop(0, ppw, row_body, 0)
                start_out(b, k)

                kn = k + PREFETCH
                bn = (b + PREFETCH) % NBUF

                @pl.when(kn < n_seq)
                def _prefetch():
                    @pl.when(k >= NBUF - PREFETCH)
                    def _drain():
                        wait_out(bn)

                    start_gather(bn, kn)

            return 0

        lax.fori_loop(0, n_seq // NBUF, body, 0)

        for b in range(NBUF):
            wait_out(b)

    return emb


def kernel(x, tok_table, pos_table):
    B, T = x.shape
    emb = _build(B, T)
    out = emb(x.reshape(B * T).astype(jnp.int32), tok_table, pos_table)
    return out.reshape(B, T, D_MODEL)
